# baseline (speedup 1.0000x reference)
import functools

import jax
import jax.numpy as jnp
from jax import lax
from jax.experimental import pallas as pl
from jax.experimental.pallas import tpu as pltpu
from jax.experimental.pallas import tpu_sc as plsc

_N = 10000
_E = 320000
_DIN = 128
_DOUT = 64
_S = 4
_B = _N // _S
_BP = 2560
_NR = 4 * _BP

_NC = 2
_NS = 16
_CH = 128
_EROWS = 2560
_EPAD = _EROWS * _CH
_RT = _EROWS // _NS
_K = 5
_K2 = 4
_PADS = 10016
_PADD = 10000

_mesh = plsc.VectorSubcoreMesh(core_axis_name="c", subcore_axis_name="s")
_sc_params = pltpu.CompilerParams(use_tc_tiling_on_sc=False)


@functools.partial(
    pl.kernel,
    out_type=(
        jax.ShapeDtypeStruct((2, _NR, 16), jnp.float32),
        jax.ShapeDtypeStruct((2, _NR, 16), jnp.float32),
    ),
    mesh=_mesh,
    scratch_types=[
        pltpu.VMEM_SHARED((_NR, 16), jnp.float32),
        pltpu.VMEM_SHARED((_NR, 16), jnp.float32),
        pltpu.VMEM((_RT, _CH), jnp.int32),
        pltpu.VMEM((_RT, _CH), jnp.int32),
        pltpu.VMEM((_CH, 16), jnp.float32),
        pltpu.VMEM((_NR // _NS, 16), jnp.float32),
        pltpu.SemaphoreType.DMA,
    ],
    compiler_params=_sc_params,
)
def _s1(sp, dp, sn, dn,
        cs_out, cd_out,
        cs, cd, sB, dB, ones_v, zbuf, sem):
    cid = lax.axis_index("c")
    sid = lax.axis_index("s")
    rr = _NR // _NS

    def fill_ones(r, _):
        ones_v[r, :] = jnp.ones((16,), jnp.float32)
        return 0
    lax.fori_loop(0, _CH, fill_ones, 0)

    def fill_z(r, _):
        zbuf[r, :] = jnp.zeros((16,), jnp.float32)
        return 0
    lax.fori_loop(0, rr, fill_z, 0)

    pltpu.sync_copy(zbuf, cs.at[pl.ds(sid * rr, rr)])
    pltpu.sync_copy(zbuf, cd.at[pl.ds(sid * rr, rr)])
    plsc.subcore_barrier()

    def run(src_e, dst_e):
        pltpu.sync_copy(src_e.at[pl.ds(sid * _RT, _RT)], sB)
        pltpu.sync_copy(dst_e.at[pl.ds(sid * _RT, _RT)], dB)

        def comp(c, _):
            for i in range(_CH // 16):
                sl = pl.ds(i * 16, 16)
                d = dB[c, sl]
                dB[c, sl] = ((d + 3) & 3) * _BP + (d >> 2)
            return 0
        lax.fori_loop(0, _RT, comp, 0)

        def grp(g, _):
            for r in range(_K):
                c = g * _K + r
                pltpu.async_copy(ones_v, cs.at[sB.at[c]], sem, add=True)
                pltpu.async_copy(ones_v, cd.at[dB.at[c]], sem, add=True)
            for r in range(_K):
                c = g * _K + r
                pltpu.make_async_copy(ones_v, cs.at[sB.at[c]], sem).wait()
                pltpu.make_async_copy(ones_v, cd.at[dB.at[c]], sem).wait()
            return 0
        lax.fori_loop(0, _RT // _K, grp, 0)

    @pl.when(cid == 0)
    def _():
        run(sp, dp)

    @pl.when(cid == 1)
    def _():
        run(sn, dn)

    plsc.subcore_barrier()
    pltpu.sync_copy(cs.at[pl.ds(sid * rr, rr)],
                    cs_out.at[cid, pl.ds(sid * rr, rr)])
    pltpu.sync_copy(cd.at[pl.ds(sid * rr, rr)],
                    cd_out.at[cid, pl.ds(sid * rr, rr)])


_T2R = 2000


def _t2_body(feat_ref, w_ref, cnt_ref, out_ref):
    cnt = cnt_ref[:, 0:1]
    scale = lax.rsqrt(jnp.maximum(cnt, 1.0))
    r = lax.broadcasted_iota(jnp.int32, (_T2R, 1), 0)
    scale = jnp.where((r % _S) == 0, 0.0, scale)
    x = feat_ref[...] * scale
    y1 = jnp.dot(x, w_ref[0], preferred_element_type=jnp.float32)
    y2 = jnp.dot(x, w_ref[1], preferred_element_type=jnp.float32)
    out_ref[...] = jnp.concatenate([y1, y2], axis=1)


def _t2(feat, wstack, cnt):
    return pl.pallas_call(
        _t2_body,
        grid=(_N // _T2R,),
        in_specs=[
            pl.BlockSpec((_T2R, _DIN), lambda i: (i, 0)),
            pl.BlockSpec((2, _DIN, _DOUT), lambda i: (0, 0, 0)),
            pl.BlockSpec((_T2R, 16), lambda i: (i, 0)),
        ],
        out_specs=pl.BlockSpec((_T2R, 2 * _DOUT), lambda i: (i, 0)),
        out_shape=jax.ShapeDtypeStruct((_N, 2 * _DOUT), jnp.float32),
    )(feat, wstack, cnt)


@functools.partial(
    pl.kernel,
    out_type=jax.ShapeDtypeStruct((2, _NR, _DOUT), jnp.float32),
    mesh=_mesh,
    scratch_types=[
        pltpu.VMEM_SHARED((_NR, _DOUT), jnp.float32),
        pltpu.VMEM((_RT // 2, _CH), jnp.int32),
        pltpu.VMEM((_RT // 2, _CH), jnp.int32),
        pltpu.VMEM((_K2 * _CH, _DOUT), jnp.float32),
        pltpu.VMEM((_K2 * _CH, _DOUT), jnp.float32),
        pltpu.SemaphoreType.DMA,
        pltpu.SemaphoreType.DMA,
        pltpu.SemaphoreType.DMA,
        pltpu.SemaphoreType.DMA,
    ],
    compiler_params=_sc_params,
)
def _s2(qp, qn, sp, dp, sn, dn,
        agg,
        acc, gB, tB, rows0, rows1, gsa, gsb, ssa, ssb):
    cid = lax.axis_index("c")
    sid = lax.axis_index("s")
    rr = _NR // _NS

    def fz(r, _):
        for i in range(_DOUT // 16):
            rows0[r, pl.ds(i * 16, 16)] = jnp.zeros((16,), jnp.float32)
        return 0
    nz = _K2 * _CH
    lax.fori_loop(0, nz, fz, 0)
    pltpu.sync_copy(rows0, acc.at[pl.ds(sid * rr, nz)])
    pltpu.sync_copy(rows0.at[pl.ds(0, rr - nz)],
                    acc.at[pl.ds(sid * rr + nz, rr - nz)])
    plsc.subcore_barrier()

    def run(q2, src_e, dst_e):
        hh = _RT // 2

        for h in range(2):
            pltpu.sync_copy(src_e.at[pl.ds(sid * _RT + h * hh, hh)], gB)
            pltpu.sync_copy(dst_e.at[pl.ds(sid * _RT + h * hh, hh)], tB)

            def comp(c, _):
                for i in range(_CH // 16):
                    sl = pl.ds(i * 16, 16)
                    s = gB[c, sl]
                    d = tB[c, sl]
                    g = s * 2 + jnp.where((d & 3) == 0, 1, 0)
                    gB[c, sl] = jnp.minimum(g, 2 * _N - 1)
                    tB[c, sl] = ((d + 3) & 3) * _BP + (d >> 2)
                return 0
            lax.fori_loop(0, hh, comp, 0)

            def grp(g, _):
                base = g * _K2
                for r in range(_K2):
                    pltpu.async_copy(q2.at[gB.at[base + r]],
                                     rows0.at[pl.ds(r * _CH, _CH)], gsa)
                for r in range(_K2):
                    pltpu.make_async_copy(q2.at[gB.at[base + r]],
                                          rows0.at[pl.ds(r * _CH, _CH)], gsa).wait()
                for r in range(_K2):
                    pltpu.async_copy(rows0.at[pl.ds(r * _CH, _CH)],
                                     acc.at[tB.at[base + r]], ssa, add=True)
                for r in range(_K2):
                    pltpu.make_async_copy(rows0.at[pl.ds(r * _CH, _CH)],
                                          acc.at[tB.at[base + r]], ssa).wait()
                return 0
            lax.fori_loop(0, hh // _K2, grp, 0)

    @pl.when(cid == 0)
    def _():
        run(qp, sp, dp)

    @pl.when(cid == 1)
    def _():
        run(qn, sn, dn)

    plsc.subcore_barrier()
    pltpu.sync_copy(acc.at[pl.ds(sid * rr, rr)],
                    agg.at[cid, pl.ds(sid * rr, rr)])


_T3G = 512


def _prelu(x, a):
    return jnp.where(x >= 0, x, a * x)


def _rownorm(x):
    return x * lax.rsqrt(jnp.maximum(jnp.sum(x * x, axis=1, keepdims=True),
                                     1e-24))


def _t3_body(pool_ref, rec_ref, pcnt_ref, rcnt_ref, anch_ref,
             w_ref, b_ref, alpha_ref, bw_ref, bb_ref,
             rdt_ref, rsc_ref):
    alpha = alpha_ref[0, 0]
    b1 = b_ref[0:1, :]
    b2 = b_ref[1:2, :]

    rec = rec_ref[0, 0]
    rc = rcnt_ref[0, 0, :, 0:1]
    rh = _prelu(rec * lax.rsqrt(jnp.maximum(rc, 1.0)) + b2, alpha)
    rn = _rownorm(rh)

    pool = jnp.zeros((_T3G, _DOUT), jnp.float32)
    for k in range(3):
        pk = pool_ref[0, k]
        ck = pcnt_ref[0, k, :, 0:1]
        pool = pool + _prelu(pk * lax.rsqrt(jnp.maximum(ck, 1.0)) + b1, alpha)
    pn = _rownorm(pool / 3.0)

    a = anch_ref[0]
    a1 = _rownorm(_prelu(jnp.dot(a, w_ref[0], preferred_element_type=jnp.float32) + b1, alpha))
    a2 = _rownorm(_prelu(jnp.dot(a, w_ref[1], preferred_element_type=jnp.float32) + b2, alpha))

    rdt_ref[0] = (jnp.sum(jnp.dot(pn, bw_ref[0], preferred_element_type=jnp.float32) * a1,
                          axis=1, keepdims=True) + bb_ref[0, 0])
    rsc_ref[0] = (jnp.sum(jnp.dot(rn, bw_ref[1], preferred_element_type=jnp.float32) * a2,
                          axis=1, keepdims=True) + bb_ref[1, 0])


def _t3(aggv, cntv, anchors, wstack, bstack, alpha, bws, bbs):
    nblk = _BP // _T3G
    return pl.pallas_call(
        _t3_body,
        grid=(2, nblk),
        in_specs=[
            pl.BlockSpec((1, 3, _T3G, _DOUT), lambda b, i: (b, 0, i, 0)),
            pl.BlockSpec((1, 1, _T3G, _DOUT), lambda b, i: (b, 3, i, 0)),
            pl.BlockSpec((1, 3, _T3G, 16), lambda b, i: (b, 0, i, 0)),
            pl.BlockSpec((1, 1, _T3G, 16), lambda b, i: (b, 3, i, 0)),
            pl.BlockSpec((1, _T3G, _DIN), lambda b, i: (b, i, 0)),
            pl.BlockSpec((2, _DIN, _DOUT), lambda b, i: (0, 0, 0)),
            pl.BlockSpec((2, _DOUT), lambda b, i: (0, 0)),
            pl.BlockSpec((1, 1), lambda b, i: (0, 0)),
            pl.BlockSpec((2, _DOUT, _DOUT), lambda b, i: (0, 0, 0)),
            pl.BlockSpec((2, 1), lambda b, i: (0, 0)),
        ],
        out_specs=[
            pl.BlockSpec((1, _T3G, 1), lambda b, i: (b, i, 0)),
            pl.BlockSpec((1, _T3G, 1), lambda b, i: (b, i, 0)),
        ],
        out_shape=[
            jax.ShapeDtypeStruct((2, _BP, 1), jnp.float32),
            jax.ShapeDtypeStruct((2, _BP, 1), jnp.float32),
        ],
    )(aggv, aggv, cntv, cntv, anchors, wstack, bstack, alpha, bws, bbs)


def kernel(pos_in_feat, pos_edge_index, neg_in_feat, neg_edge_index,
           weight1, weight2, bias1, bias2, prelu_alpha,
           bil_w1, bil_b1, bil_w2, bil_b2):
    npad = _EPAD - _E
    sp = jnp.pad(pos_edge_index[0], (0, npad),
                 constant_values=_PADS).reshape(_EROWS, _CH)
    dp = jnp.pad(pos_edge_index[1], (0, npad),
                 constant_values=_PADD).reshape(_EROWS, _CH)
    sn = jnp.pad(neg_edge_index[0], (0, npad),
                 constant_values=_PADS).reshape(_EROWS, _CH)
    dn = jnp.pad(neg_edge_index[1], (0, npad),
                 constant_values=_PADD).reshape(_EROWS, _CH)

    cs, cd = _s1(sp, dp, sn, dn)

    wstack = jnp.stack([weight1, weight2])
    qp = _t2(pos_in_feat, wstack, cs[0])
    qn = _t2(neg_in_feat, wstack, cs[1])

    agg = _s2(qp.reshape(2 * _N, _DOUT), qn.reshape(2 * _N, _DOUT),
              sp, dp, sn, dn)

    aggv = agg.reshape(2, 4, _BP, _DOUT)
    cntv = cd.reshape(2, 4, _BP, 16)

    anch = jnp.stack([pos_in_feat, neg_in_feat]).reshape(2, _B, _S, _DIN)[:, :, 0, :]
    anch = jnp.pad(anch, ((0, 0), (0, _BP - _B), (0, 0)))

    bstack = jnp.stack([bias1, bias2])
    alpha = prelu_alpha.reshape(1, 1).astype(jnp.float32)
    bws = jnp.concatenate([bil_w1, bil_w2], axis=0)
    bbs = jnp.stack([bil_b1, bil_b2])

    rdt, rsc = _t3(aggv, cntv, anch, wstack, bstack, alpha, bws, bbs)

    return (rdt[0, :_B], rsc[0, :_B], rdt[1, :_B], rsc[1, :_B])


# speedup vs baseline: 1.6599x; 1.6599x over previous
import functools

import jax
import jax.numpy as jnp
from jax import lax
from jax.experimental import pallas as pl
from jax.experimental.pallas import tpu as pltpu
from jax.experimental.pallas import tpu_sc as plsc

_N = 10000
_E = 320000
_DIN = 128
_DOUT = 64
_S = 4
_B = _N // _S
_BP = 2560
_NR = 4 * _BP

_NC = 2
_NS = 16
_CH = 80
_RT = _E // _NS // _CH
_K = 5
_K2 = 5

_mesh = plsc.VectorSubcoreMesh(core_axis_name="c", subcore_axis_name="s")
_sc_params = pltpu.CompilerParams(use_tc_tiling_on_sc=False)


@functools.partial(
    pl.kernel,
    out_type=(
        jax.ShapeDtypeStruct((2, _NR, 16), jnp.float32),
        jax.ShapeDtypeStruct((2, _NR, 16), jnp.float32),
    ),
    mesh=_mesh,
    scratch_types=[
        pltpu.VMEM_SHARED((_NR, 16), jnp.float32),
        pltpu.VMEM_SHARED((_NR, 16), jnp.float32),
        pltpu.VMEM((_RT, _CH), jnp.int32),
        pltpu.VMEM((_RT, _CH), jnp.int32),
        pltpu.VMEM((_CH, 16), jnp.float32),
        pltpu.VMEM((_NR // _NS, 16), jnp.float32),
        pltpu.SemaphoreType.DMA,
    ],
    compiler_params=_sc_params,
)
def _s1(sp, dp, sn, dn,
        cs_out, cd_out,
        cs, cd, sB, dB, ones_v, zbuf, sem):
    cid = lax.axis_index("c")
    sid = lax.axis_index("s")
    rr = _NR // _NS

    def fill_ones(r, _):
        ones_v[r, :] = jnp.ones((16,), jnp.float32)
        return 0
    lax.fori_loop(0, _CH, fill_ones, 0)

    def fill_z(r, _):
        zbuf[r, :] = jnp.zeros((16,), jnp.float32)
        return 0
    lax.fori_loop(0, rr, fill_z, 0)

    pltpu.sync_copy(zbuf, cs.at[pl.ds(sid * rr, rr)])
    pltpu.sync_copy(zbuf, cd.at[pl.ds(sid * rr, rr)])
    plsc.subcore_barrier()

    def run(src_e, dst_e):
        pltpu.sync_copy(src_e.at[pl.ds(sid * _RT, _RT)], sB)
        pltpu.sync_copy(dst_e.at[pl.ds(sid * _RT, _RT)], dB)

        def comp(c, _):
            for i in range(_CH // 16):
                sl = pl.ds(i * 16, 16)
                d = dB[c, sl]
                dB[c, sl] = ((d + 3) & 3) * _BP + (d >> 2)
            return 0
        lax.fori_loop(0, _RT, comp, 0)

        def grp(g, _):
            for r in range(_K):
                c = g * _K + r
                pltpu.async_copy(ones_v, cs.at[sB.at[c]], sem, add=True)
                pltpu.async_copy(ones_v, cd.at[dB.at[c]], sem, add=True)
            for r in range(_K):
                c = g * _K + r
                pltpu.make_async_copy(ones_v, cs.at[sB.at[c]], sem).wait()
                pltpu.make_async_copy(ones_v, cd.at[dB.at[c]], sem).wait()
            return 0
        lax.fori_loop(0, _RT // _K, grp, 0)

    @pl.when(cid == 0)
    def _():
        run(sp, dp)

    @pl.when(cid == 1)
    def _():
        run(sn, dn)

    plsc.subcore_barrier()
    pltpu.sync_copy(cs.at[pl.ds(sid * rr, rr)],
                    cs_out.at[cid, pl.ds(sid * rr, rr)])
    pltpu.sync_copy(cd.at[pl.ds(sid * rr, rr)],
                    cd_out.at[cid, pl.ds(sid * rr, rr)])


_T2R = 2000


def _t2_body(feat_ref, w_ref, cnt_ref, out_ref):
    cnt = cnt_ref[:, 0:1]
    scale = lax.rsqrt(jnp.maximum(cnt, 1.0))
    r = lax.broadcasted_iota(jnp.int32, (_T2R, 1), 0)
    scale = jnp.where((r % _S) == 0, 0.0, scale)
    x = feat_ref[...] * scale
    y1 = jnp.dot(x, w_ref[0], preferred_element_type=jnp.float32)
    y2 = jnp.dot(x, w_ref[1], preferred_element_type=jnp.float32)
    out_ref[...] = jnp.concatenate([y1, y2], axis=1)


def _t2(feat, wstack, cnt):
    return pl.pallas_call(
        _t2_body,
        grid=(_N // _T2R,),
        in_specs=[
            pl.BlockSpec((_T2R, _DIN), lambda i: (i, 0)),
            pl.BlockSpec((2, _DIN, _DOUT), lambda i: (0, 0, 0)),
            pl.BlockSpec((_T2R, 16), lambda i: (i, 0)),
        ],
        out_specs=pl.BlockSpec((_T2R, 2 * _DOUT), lambda i: (i, 0)),
        out_shape=jax.ShapeDtypeStruct((_N, 2 * _DOUT), jnp.float32),
    )(feat, wstack, cnt)


@functools.partial(
    pl.kernel,
    out_type=jax.ShapeDtypeStruct((2, _NR, _DOUT), jnp.float32),
    mesh=_mesh,
    scratch_types=[
        pltpu.VMEM_SHARED((_NR, _DOUT), jnp.float32),
        pltpu.VMEM((_RT // 2, _CH), jnp.int32),
        pltpu.VMEM((_RT // 2, _CH), jnp.int32),
        pltpu.VMEM((_K2 * _CH, _DOUT), jnp.float32),
        pltpu.SemaphoreType.DMA,
        pltpu.SemaphoreType.DMA,
        pltpu.SemaphoreType.DMA,
        pltpu.SemaphoreType.DMA,
    ],
    compiler_params=_sc_params,
)
def _s2(qp, qn, sp, dp, sn, dn,
        agg,
        acc, gB, tB, rows0, gsa, gsb, ssa, ssb):
    cid = lax.axis_index("c")
    sid = lax.axis_index("s")
    rr = _NR // _NS

    def fz(r, _):
        for i in range(_DOUT // 16):
            rows0[r, pl.ds(i * 16, 16)] = jnp.zeros((16,), jnp.float32)
        return 0
    nz = _K2 * _CH
    lax.fori_loop(0, nz, fz, 0)
    pltpu.sync_copy(rows0, acc.at[pl.ds(sid * rr, nz)])
    pltpu.sync_copy(rows0.at[pl.ds(0, rr - nz)],
                    acc.at[pl.ds(sid * rr + nz, rr - nz)])
    plsc.subcore_barrier()

    def run(q2, src_e, dst_e):
        hh = _RT // 2

        for h in range(2):
            pltpu.sync_copy(src_e.at[pl.ds(sid * _RT + h * hh, hh)], gB)
            pltpu.sync_copy(dst_e.at[pl.ds(sid * _RT + h * hh, hh)], tB)

            def comp(c, _):
                for i in range(_CH // 16):
                    sl = pl.ds(i * 16, 16)
                    s = gB[c, sl]
                    d = tB[c, sl]
                    g = s * 2 + jnp.where((d & 3) == 0, 1, 0)
                    gB[c, sl] = jnp.minimum(g, 2 * _N - 1)
                    tB[c, sl] = ((d + 3) & 3) * _BP + (d >> 2)
                return 0
            lax.fori_loop(0, hh, comp, 0)

            def grp(g, _):
                base = g * _K2
                for r in range(_K2):
                    pltpu.async_copy(q2.at[gB.at[base + r]],
                                     rows0.at[pl.ds(r * _CH, _CH)], gsa)
                for r in range(_K2):
                    pltpu.make_async_copy(q2.at[gB.at[base + r]],
                                          rows0.at[pl.ds(r * _CH, _CH)], gsa).wait()
                for r in range(_K2):
                    pltpu.async_copy(rows0.at[pl.ds(r * _CH, _CH)],
                                     acc.at[tB.at[base + r]], ssa, add=True)
                for r in range(_K2):
                    pltpu.make_async_copy(rows0.at[pl.ds(r * _CH, _CH)],
                                          acc.at[tB.at[base + r]], ssa).wait()
                return 0
            lax.fori_loop(0, hh // _K2, grp, 0)

    @pl.when(cid == 0)
    def _():
        run(qp, sp, dp)

    @pl.when(cid == 1)
    def _():
        run(qn, sn, dn)

    plsc.subcore_barrier()
    pltpu.sync_copy(acc.at[pl.ds(sid * rr, rr)],
                    agg.at[cid, pl.ds(sid * rr, rr)])


_T3G = 512


def _prelu(x, a):
    return jnp.where(x >= 0, x, a * x)


def _rownorm(x):
    return x * lax.rsqrt(jnp.maximum(jnp.sum(x * x, axis=1, keepdims=True),
                                     1e-24))


def _t3_body(pool_ref, rec_ref, pcnt_ref, rcnt_ref, anch_ref,
             w_ref, b_ref, alpha_ref, bw_ref, bb_ref,
             rdt_ref, rsc_ref):
    alpha = alpha_ref[0, 0]
    b1 = b_ref[0:1, :]
    b2 = b_ref[1:2, :]

    rec = rec_ref[0, 0]
    rc = rcnt_ref[0, 0, :, 0:1]
    rh = _prelu(rec * lax.rsqrt(jnp.maximum(rc, 1.0)) + b2, alpha)
    rn = _rownorm(rh)

    pool = jnp.zeros((_T3G, _DOUT), jnp.float32)
    for k in range(3):
        pk = pool_ref[0, k]
        ck = pcnt_ref[0, k, :, 0:1]
        pool = pool + _prelu(pk * lax.rsqrt(jnp.maximum(ck, 1.0)) + b1, alpha)
    pn = _rownorm(pool / 3.0)

    a = anch_ref[0]
    a1 = _rownorm(_prelu(jnp.dot(a, w_ref[0], preferred_element_type=jnp.float32) + b1, alpha))
    a2 = _rownorm(_prelu(jnp.dot(a, w_ref[1], preferred_element_type=jnp.float32) + b2, alpha))

    rdt_ref[0] = (jnp.sum(jnp.dot(pn, bw_ref[0], preferred_element_type=jnp.float32) * a1,
                          axis=1, keepdims=True) + bb_ref[0, 0])
    rsc_ref[0] = (jnp.sum(jnp.dot(rn, bw_ref[1], preferred_element_type=jnp.float32) * a2,
                          axis=1, keepdims=True) + bb_ref[1, 0])


def _t3(aggv, cntv, anchors, wstack, bstack, alpha, bws, bbs):
    nblk = _BP // _T3G
    return pl.pallas_call(
        _t3_body,
        grid=(2, nblk),
        in_specs=[
            pl.BlockSpec((1, 3, _T3G, _DOUT), lambda b, i: (b, 0, i, 0)),
            pl.BlockSpec((1, 1, _T3G, _DOUT), lambda b, i: (b, 3, i, 0)),
            pl.BlockSpec((1, 3, _T3G, 16), lambda b, i: (b, 0, i, 0)),
            pl.BlockSpec((1, 1, _T3G, 16), lambda b, i: (b, 3, i, 0)),
            pl.BlockSpec((1, _T3G, _DIN), lambda b, i: (b, i, 0)),
            pl.BlockSpec((2, _DIN, _DOUT), lambda b, i: (0, 0, 0)),
            pl.BlockSpec((2, _DOUT), lambda b, i: (0, 0)),
            pl.BlockSpec((1, 1), lambda b, i: (0, 0)),
            pl.BlockSpec((2, _DOUT, _DOUT), lambda b, i: (0, 0, 0)),
            pl.BlockSpec((2, 1), lambda b, i: (0, 0)),
        ],
        out_specs=[
            pl.BlockSpec((1, _T3G, 1), lambda b, i: (b, i, 0)),
            pl.BlockSpec((1, _T3G, 1), lambda b, i: (b, i, 0)),
        ],
        out_shape=[
            jax.ShapeDtypeStruct((2, _BP, 1), jnp.float32),
            jax.ShapeDtypeStruct((2, _BP, 1), jnp.float32),
        ],
    )(aggv, aggv, cntv, cntv, anchors, wstack, bstack, alpha, bws, bbs)


def kernel(pos_in_feat, pos_edge_index, neg_in_feat, neg_edge_index,
           weight1, weight2, bias1, bias2, prelu_alpha,
           bil_w1, bil_b1, bil_w2, bil_b2):
    sp = pos_edge_index[0].reshape(_NS * _RT, _CH)
    dp = pos_edge_index[1].reshape(_NS * _RT, _CH)
    sn = neg_edge_index[0].reshape(_NS * _RT, _CH)
    dn = neg_edge_index[1].reshape(_NS * _RT, _CH)

    cs, cd = _s1(sp, dp, sn, dn)

    wstack = jnp.stack([weight1, weight2])
    qp = _t2(pos_in_feat, wstack, cs[0])
    qn = _t2(neg_in_feat, wstack, cs[1])

    agg = _s2(qp.reshape(2 * _N, _DOUT), qn.reshape(2 * _N, _DOUT),
              sp, dp, sn, dn)

    aggv = agg.reshape(2, 4, _BP, _DOUT)
    cntv = cd.reshape(2, 4, _BP, 16)

    anch = jnp.stack([pos_in_feat, neg_in_feat]).reshape(2, _B, _S, _DIN)[:, :, 0, :]
    anch = jnp.pad(anch, ((0, 0), (0, _BP - _B), (0, 0)))

    bstack = jnp.stack([bias1, bias2])
    alpha = prelu_alpha.reshape(1, 1).astype(jnp.float32)
    bws = jnp.concatenate([bil_w1, bil_w2], axis=0)
    bbs = jnp.stack([bil_b1, bil_b2])

    rdt, rsc = _t3(aggv, cntv, anch, wstack, bstack, alpha, bws, bbs)

    return (rdt[0, :_B], rsc[0, :_B], rdt[1, :_B], rsc[1, :_B])


# speedup vs baseline: 1.7141x; 1.0327x over previous
import functools

import jax
import jax.numpy as jnp
from jax import lax
from jax.experimental import pallas as pl
from jax.experimental.pallas import tpu as pltpu
from jax.experimental.pallas import tpu_sc as plsc

_N = 10000
_E = 320000
_DIN = 128
_DOUT = 64
_S = 4
_B = _N // _S
_BP = 2560
_NR = 4 * _BP

_NC = 2
_NS = 16
_CH = 80
_RT = _E // _NS // _CH
_K = 5
_K2 = 5

_mesh = plsc.VectorSubcoreMesh(core_axis_name="c", subcore_axis_name="s")
_sc_params = pltpu.CompilerParams(use_tc_tiling_on_sc=False)


@functools.partial(
    pl.kernel,
    out_type=(
        jax.ShapeDtypeStruct((2, _NR, 16), jnp.float32),
        jax.ShapeDtypeStruct((2, _NR, 16), jnp.float32),
    ),
    mesh=_mesh,
    scratch_types=[
        pltpu.VMEM_SHARED((_NR, 16), jnp.float32),
        pltpu.VMEM_SHARED((_NR, 16), jnp.float32),
        pltpu.VMEM((_RT, _CH), jnp.int32),
        pltpu.VMEM((_RT, _CH), jnp.int32),
        pltpu.VMEM((_CH, 16), jnp.float32),
        pltpu.VMEM((_NR // _NS, 16), jnp.float32),
        pltpu.SemaphoreType.DMA,
    ],
    compiler_params=_sc_params,
)
def _s1(sp, dp, sn, dn,
        cs_out, cd_out,
        cs, cd, sB, dB, ones_v, zbuf, sem):
    cid = lax.axis_index("c")
    sid = lax.axis_index("s")
    rr = _NR // _NS

    def fill_ones(r, _):
        ones_v[r, :] = jnp.ones((16,), jnp.float32)
        return 0
    lax.fori_loop(0, _CH, fill_ones, 0)

    def fill_z(r, _):
        zbuf[r, :] = jnp.zeros((16,), jnp.float32)
        return 0
    lax.fori_loop(0, rr, fill_z, 0)

    pltpu.sync_copy(zbuf, cs.at[pl.ds(sid * rr, rr)])
    pltpu.sync_copy(zbuf, cd.at[pl.ds(sid * rr, rr)])
    plsc.subcore_barrier()

    def run(src_e, dst_e):
        pltpu.sync_copy(src_e.at[pl.ds(sid * _RT, _RT)], sB)
        pltpu.sync_copy(dst_e.at[pl.ds(sid * _RT, _RT)], dB)

        def comp(c, _):
            for i in range(_CH // 16):
                sl = pl.ds(i * 16, 16)
                d = dB[c, sl]
                dB[c, sl] = ((d + 3) & 3) * _BP + (d >> 2)
            return 0
        lax.fori_loop(0, _RT, comp, 0)

        def grp(g, _):
            for r in range(_K):
                c = g * _K + r
                pltpu.async_copy(ones_v, cs.at[sB.at[c]], sem, add=True)
                pltpu.async_copy(ones_v, cd.at[dB.at[c]], sem, add=True)
            for r in range(_K):
                c = g * _K + r
                pltpu.make_async_copy(ones_v, cs.at[sB.at[c]], sem).wait()
                pltpu.make_async_copy(ones_v, cd.at[dB.at[c]], sem).wait()
            return 0
        lax.fori_loop(0, _RT // _K, grp, 0)

    @pl.when(cid == 0)
    def _():
        run(sp, dp)

    @pl.when(cid == 1)
    def _():
        run(sn, dn)

    plsc.subcore_barrier()
    pltpu.sync_copy(cs.at[pl.ds(sid * rr, rr)],
                    cs_out.at[cid, pl.ds(sid * rr, rr)])
    pltpu.sync_copy(cd.at[pl.ds(sid * rr, rr)],
                    cd_out.at[cid, pl.ds(sid * rr, rr)])


_T2R = 2000


def _t2_body(feat_ref, w_ref, cnt_ref, out_ref):
    cnt = cnt_ref[:, 0:1]
    scale = lax.rsqrt(jnp.maximum(cnt, 1.0))
    r = lax.broadcasted_iota(jnp.int32, (_T2R, 1), 0)
    scale = jnp.where((r % _S) == 0, 0.0, scale)
    x = feat_ref[...] * scale
    y1 = jnp.dot(x, w_ref[0], preferred_element_type=jnp.float32)
    y2 = jnp.dot(x, w_ref[1], preferred_element_type=jnp.float32)
    out_ref[...] = jnp.concatenate([y1, y2], axis=1)


def _t2(feat, wstack, cnt):
    return pl.pallas_call(
        _t2_body,
        grid=(_N // _T2R,),
        in_specs=[
            pl.BlockSpec((_T2R, _DIN), lambda i: (i, 0)),
            pl.BlockSpec((2, _DIN, _DOUT), lambda i: (0, 0, 0)),
            pl.BlockSpec((_T2R, 16), lambda i: (i, 0)),
        ],
        out_specs=pl.BlockSpec((_T2R, 2 * _DOUT), lambda i: (i, 0)),
        out_shape=jax.ShapeDtypeStruct((_N, 2 * _DOUT), jnp.float32),
    )(feat, wstack, cnt)


@functools.partial(
    pl.kernel,
    out_type=jax.ShapeDtypeStruct((2, _NR, _DOUT), jnp.float32),
    mesh=_mesh,
    scratch_types=[
        pltpu.VMEM_SHARED((_NR, _DOUT), jnp.float32),
        pltpu.VMEM((150, _CH), jnp.int32),
        pltpu.VMEM((150, _CH), jnp.int32),
        pltpu.VMEM((_K2 * _CH, _DOUT), jnp.float32),
        pltpu.VMEM((_K2 * _CH, _DOUT), jnp.float32),
        pltpu.SemaphoreType.DMA,
        pltpu.SemaphoreType.DMA,
        pltpu.SemaphoreType.DMA,
        pltpu.SemaphoreType.DMA,
    ],
    compiler_params=_sc_params,
)
def _s2(qp, qn, sp, dp, sn, dn,
        agg,
        acc, gB, tB, rows0, rows1, gsa, gsb, ssa, ssb):
    cid = lax.axis_index("c")
    sid = lax.axis_index("s")
    rr = _NR // _NS

    def fz(r, _):
        for i in range(_DOUT // 16):
            rows0[r, pl.ds(i * 16, 16)] = jnp.zeros((16,), jnp.float32)
        return 0
    nz = _K2 * _CH
    lax.fori_loop(0, nz, fz, 0)
    pltpu.sync_copy(rows0, acc.at[pl.ds(sid * rr, nz)])
    pltpu.sync_copy(rows0.at[pl.ds(0, rr - nz)],
                    acc.at[pl.ds(sid * rr + nz, rr - nz)])
    plsc.subcore_barrier()

    def run(q2, src_e, dst_e):
        for h0, hh in ((0, 150), (150, 100)):
            pltpu.sync_copy(src_e.at[pl.ds(sid * _RT + h0, hh)],
                            gB.at[pl.ds(0, hh)])
            pltpu.sync_copy(dst_e.at[pl.ds(sid * _RT + h0, hh)],
                            tB.at[pl.ds(0, hh)])

            def comp(c, _):
                for i in range(_CH // 16):
                    sl = pl.ds(i * 16, 16)
                    s = gB[c, sl]
                    d = tB[c, sl]
                    g = s * 2 + jnp.where((d & 3) == 0, 1, 0)
                    gB[c, sl] = jnp.minimum(g, 2 * _N - 1)
                    tB[c, sl] = ((d + 3) & 3) * _BP + (d >> 2)
                return 0
            lax.fori_loop(0, hh, comp, 0)

            def pair(t, _):
                base = t * (2 * _K2)
                for r in range(_K2):
                    pltpu.async_copy(q2.at[gB.at[base + r]],
                                     rows0.at[pl.ds(r * _CH, _CH)], gsa)
                for r in range(_K2):
                    pltpu.async_copy(q2.at[gB.at[base + _K2 + r]],
                                     rows1.at[pl.ds(r * _CH, _CH)], gsb)
                for r in range(_K2):
                    pltpu.make_async_copy(q2.at[gB.at[base + r]],
                                          rows0.at[pl.ds(r * _CH, _CH)], gsa).wait()
                for r in range(_K2):
                    pltpu.async_copy(rows0.at[pl.ds(r * _CH, _CH)],
                                     acc.at[tB.at[base + r]], ssa, add=True)
                for r in range(_K2):
                    pltpu.make_async_copy(q2.at[gB.at[base + _K2 + r]],
                                          rows1.at[pl.ds(r * _CH, _CH)], gsb).wait()
                for r in range(_K2):
                    pltpu.async_copy(rows1.at[pl.ds(r * _CH, _CH)],
                                     acc.at[tB.at[base + _K2 + r]], ssb, add=True)
                for r in range(_K2):
                    pltpu.make_async_copy(rows0.at[pl.ds(r * _CH, _CH)],
                                          acc.at[tB.at[base + r]], ssa).wait()
                for r in range(_K2):
                    pltpu.make_async_copy(rows1.at[pl.ds(r * _CH, _CH)],
                                          acc.at[tB.at[base + _K2 + r]], ssb).wait()
                return 0
            lax.fori_loop(0, hh // (2 * _K2), pair, 0)

    @pl.when(cid == 0)
    def _():
        run(qp, sp, dp)

    @pl.when(cid == 1)
    def _():
        run(qn, sn, dn)

    plsc.subcore_barrier()
    pltpu.sync_copy(acc.at[pl.ds(sid * rr, rr)],
                    agg.at[cid, pl.ds(sid * rr, rr)])


_T3G = 512


def _prelu(x, a):
    return jnp.where(x >= 0, x, a * x)


def _rownorm(x):
    return x * lax.rsqrt(jnp.maximum(jnp.sum(x * x, axis=1, keepdims=True),
                                     1e-24))


def _t3_body(pool_ref, rec_ref, pcnt_ref, rcnt_ref, anch_ref,
             w_ref, b_ref, alpha_ref, bw_ref, bb_ref,
             rdt_ref, rsc_ref):
    alpha = alpha_ref[0, 0]
    b1 = b_ref[0:1, :]
    b2 = b_ref[1:2, :]

    rec = rec_ref[0, 0]
    rc = rcnt_ref[0, 0, :, 0:1]
    rh = _prelu(rec * lax.rsqrt(jnp.maximum(rc, 1.0)) + b2, alpha)
    rn = _rownorm(rh)

    pool = jnp.zeros((_T3G, _DOUT), jnp.float32)
    for k in range(3):
        pk = pool_ref[0, k]
        ck = pcnt_ref[0, k, :, 0:1]
        pool = pool + _prelu(pk * lax.rsqrt(jnp.maximum(ck, 1.0)) + b1, alpha)
    pn = _rownorm(pool / 3.0)

    a = anch_ref[0]
    a1 = _rownorm(_prelu(jnp.dot(a, w_ref[0], preferred_element_type=jnp.float32) + b1, alpha))
    a2 = _rownorm(_prelu(jnp.dot(a, w_ref[1], preferred_element_type=jnp.float32) + b2, alpha))

    rdt_ref[0] = (jnp.sum(jnp.dot(pn, bw_ref[0], preferred_element_type=jnp.float32) * a1,
                          axis=1, keepdims=True) + bb_ref[0, 0])
    rsc_ref[0] = (jnp.sum(jnp.dot(rn, bw_ref[1], preferred_element_type=jnp.float32) * a2,
                          axis=1, keepdims=True) + bb_ref[1, 0])


def _t3(aggv, cntv, anchors, wstack, bstack, alpha, bws, bbs):
    nblk = _BP // _T3G
    return pl.pallas_call(
        _t3_body,
        grid=(2, nblk),
        in_specs=[
            pl.BlockSpec((1, 3, _T3G, _DOUT), lambda b, i: (b, 0, i, 0)),
            pl.BlockSpec((1, 1, _T3G, _DOUT), lambda b, i: (b, 3, i, 0)),
            pl.BlockSpec((1, 3, _T3G, 16), lambda b, i: (b, 0, i, 0)),
            pl.BlockSpec((1, 1, _T3G, 16), lambda b, i: (b, 3, i, 0)),
            pl.BlockSpec((1, _T3G, _DIN), lambda b, i: (b, i, 0)),
            pl.BlockSpec((2, _DIN, _DOUT), lambda b, i: (0, 0, 0)),
            pl.BlockSpec((2, _DOUT), lambda b, i: (0, 0)),
            pl.BlockSpec((1, 1), lambda b, i: (0, 0)),
            pl.BlockSpec((2, _DOUT, _DOUT), lambda b, i: (0, 0, 0)),
            pl.BlockSpec((2, 1), lambda b, i: (0, 0)),
        ],
        out_specs=[
            pl.BlockSpec((1, _T3G, 1), lambda b, i: (b, i, 0)),
            pl.BlockSpec((1, _T3G, 1), lambda b, i: (b, i, 0)),
        ],
        out_shape=[
            jax.ShapeDtypeStruct((2, _BP, 1), jnp.float32),
            jax.ShapeDtypeStruct((2, _BP, 1), jnp.float32),
        ],
    )(aggv, aggv, cntv, cntv, anchors, wstack, bstack, alpha, bws, bbs)


def kernel(pos_in_feat, pos_edge_index, neg_in_feat, neg_edge_index,
           weight1, weight2, bias1, bias2, prelu_alpha,
           bil_w1, bil_b1, bil_w2, bil_b2):
    sp = pos_edge_index[0].reshape(_NS * _RT, _CH)
    dp = pos_edge_index[1].reshape(_NS * _RT, _CH)
    sn = neg_edge_index[0].reshape(_NS * _RT, _CH)
    dn = neg_edge_index[1].reshape(_NS * _RT, _CH)

    cs, cd = _s1(sp, dp, sn, dn)

    wstack = jnp.stack([weight1, weight2])
    qp = _t2(pos_in_feat, wstack, cs[0])
    qn = _t2(neg_in_feat, wstack, cs[1])

    agg = _s2(qp.reshape(2 * _N, _DOUT), qn.reshape(2 * _N, _DOUT),
              sp, dp, sn, dn)

    aggv = agg.reshape(2, 4, _BP, _DOUT)
    cntv = cd.reshape(2, 4, _BP, 16)

    anch = jnp.stack([pos_in_feat, neg_in_feat]).reshape(2, _B, _S, _DIN)[:, :, 0, :]
    anch = jnp.pad(anch, ((0, 0), (0, _BP - _B), (0, 0)))

    bstack = jnp.stack([bias1, bias2])
    alpha = prelu_alpha.reshape(1, 1).astype(jnp.float32)
    bws = jnp.concatenate([bil_w1, bil_w2], axis=0)
    bbs = jnp.stack([bil_b1, bil_b2])

    rdt, rsc = _t3(aggv, cntv, anch, wstack, bstack, alpha, bws, bbs)

    return (rdt[0, :_B], rsc[0, :_B], rdt[1, :_B], rsc[1, :_B])
